# Initial kernel scaffold; baseline (speedup 1.0000x reference)
import jax
import jax.numpy as jnp
from jax.experimental import pallas as pl


def kernel(x, edge_index):
    raise NotImplementedError("write your pallas kernel here")


# speedup vs baseline: 5.7560x; 5.7560x over previous
import functools

import jax
import jax.numpy as jnp
from jax import lax
from jax.experimental import pallas as pl
from jax.experimental.pallas import tpu as pltpu
from jax.experimental.pallas import tpu_sc as plsc

N = 10000
D = 128
TRASH = 240
NP = N + TRASH
NTILES = 32
RPT = NP // 16
CHUNK = 128

@functools.cache
def _mesh():
    return plsc.VectorSubcoreMesh(core_axis_name="c", subcore_axis_name="s",
                                  num_cores=2, num_subcores=16)


def _agg_body(with_deg, streams, per_tile, *refs):
    if with_deg:
        (in_h, src_h, dst_h, z2_h, z1_h, one_h,
         p0_h, p1_h, d0_h, d1_h,
         acc, dacc, sidx, didx, rows, ones_v, sem) = refs
    else:
        (in_h, src_h, dst_h, z2_h,
         p0_h, p1_h,
         acc, sidx, didx, rows, sem) = refs
    c = lax.axis_index("c")
    s = lax.axis_index("s")
    rbase = s * RPT
    pltpu.sync_copy(z2_h, acc.at[pl.ds(rbase, RPT)])
    if with_deg:
        pltpu.sync_copy(z1_h, dacc.at[pl.ds(rbase, RPT)])
        pltpu.sync_copy(one_h, ones_v)
    plsc.subcore_barrier()

    ebase = (c * 16 + s) * per_tile

    @pl.loop(0, streams)
    def _(k):
        off = ebase + k * CHUNK
        pltpu.sync_copy(src_h.at[pl.ds(off, CHUNK)], sidx)
        pltpu.sync_copy(dst_h.at[pl.ds(off, CHUNK)], didx)
        pltpu.async_copy(in_h.at[sidx], rows, sem).wait()
        pltpu.sync_copy(rows, acc.at[didx], add=True)
        if with_deg:
            pltpu.sync_copy(ones_v, dacc.at[didx], add=True)

    plsc.subcore_barrier()

    @pl.when(c == 0)
    def _():
        pltpu.sync_copy(acc.at[pl.ds(rbase, RPT)], p0_h.at[pl.ds(rbase, RPT)])
        if with_deg:
            pltpu.sync_copy(dacc.at[pl.ds(rbase, RPT)], d0_h.at[pl.ds(rbase, RPT)])

    @pl.when(c == 1)
    def _():
        pltpu.sync_copy(acc.at[pl.ds(rbase, RPT)], p1_h.at[pl.ds(rbase, RPT)])
        if with_deg:
            pltpu.sync_copy(dacc.at[pl.ds(rbase, RPT)], d1_h.at[pl.ds(rbase, RPT)])


def _make_agg(with_deg, streams, per_tile):
    f32 = jnp.float32
    out_type = [jax.ShapeDtypeStruct((NP, D), f32),
                jax.ShapeDtypeStruct((NP, D), f32)]
    scratch = [pltpu.VMEM_SHARED((NP, D), f32)]
    if with_deg:
        out_type += [jax.ShapeDtypeStruct((NP,), f32),
                     jax.ShapeDtypeStruct((NP,), f32)]
        scratch += [pltpu.VMEM_SHARED((NP,), f32)]
    scratch += [pltpu.VMEM((CHUNK,), jnp.int32),
                pltpu.VMEM((CHUNK,), jnp.int32),
                pltpu.VMEM((CHUNK, D), f32)]
    if with_deg:
        scratch += [pltpu.VMEM((CHUNK,), f32)]
    scratch += [pltpu.SemaphoreType.DMA]
    return pl.kernel(
        functools.partial(_agg_body, with_deg, streams, per_tile),
        out_type=out_type,
        mesh=_mesh(),
        scratch_types=scratch,
    )


def _combine_body(p0, p1, d0, d1, out):
    p = p0[...] + p1[...]
    d = d0[...] + d1[...]
    inv = 1.0 / jnp.maximum(d, 1.0)
    p3 = p.reshape(8, 128, D)
    out[...] = jnp.maximum(p3 * inv[:, :, None], 0.0).reshape(1024, D)


def _combine(P0, P1, d0r, d1r):
    return pl.pallas_call(
        _combine_body,
        grid=(NP // 1024,),
        in_specs=[
            pl.BlockSpec((1024, D), lambda i: (i, 0)),
            pl.BlockSpec((1024, D), lambda i: (i, 0)),
            pl.BlockSpec((8, 128), lambda i: (i, 0)),
            pl.BlockSpec((8, 128), lambda i: (i, 0)),
        ],
        out_specs=pl.BlockSpec((1024, D), lambda i: (i, 0)),
        out_shape=jax.ShapeDtypeStruct((NP, D), jnp.float32),
    )(P0, P1, d0r, d1r)


def _mm_body(a, b, out):
    out[...] = lax.dot_general(a[...], b[...], (((1,), (1,)), ((), ())),
                               preferred_element_type=jnp.float32)


def _matmul(s_p):
    bm, bn = 512, 512
    return pl.pallas_call(
        _mm_body,
        grid=(NP // bm, NP // bn),
        in_specs=[
            pl.BlockSpec((bm, D), lambda i, j: (i, 0)),
            pl.BlockSpec((bn, D), lambda i, j: (j, 0)),
        ],
        out_specs=pl.BlockSpec((bm, bn), lambda i, j: (i, j)),
        out_shape=jax.ShapeDtypeStruct((N, N), jnp.float32),
    )(s_p, s_p)


def kernel(x, edge_index):
    f32 = jnp.float32
    src = edge_index[0].astype(jnp.int32)
    dst = edge_index[1].astype(jnp.int32)
    e = src.shape[0]
    per = NTILES * CHUNK
    ep = ((e + per - 1) // per) * per
    padn = ep - e
    pad_ids = jnp.arange(padn, dtype=jnp.int32)
    srcp = jnp.concatenate([src, (pad_ids * 37) % N])
    dstp = jnp.concatenate([dst, N + pad_ids % TRASH])
    per_tile = ep // NTILES
    streams = per_tile // CHUNK

    z2 = jnp.zeros((RPT, D), f32)
    z1 = jnp.zeros((RPT,), f32)
    one = jnp.ones((CHUNK,), f32)

    agg1 = _make_agg(True, streams, per_tile)
    agg = _make_agg(False, streams, per_tile)

    P0, P1, D0, D1 = agg1(x, srcp, dstp, z2, z1, one)
    d0r = D0.reshape(NP // 128, 128)
    d1r = D1.reshape(NP // 128, 128)
    h1 = _combine(P0, P1, d0r, d1r)
    Q0, Q1 = agg(h1, srcp, dstp, z2)
    h2 = _combine(Q0, Q1, d0r, d1r)
    R0, R1 = agg(h2, srcp, dstp, z2)
    s_p = _combine(R0, R1, d0r, d1r)
    T0, T1 = agg(s_p, srcp, dstp, z2)
    xh_p = _combine(T0, T1, d0r, d1r)
    a_hat = _matmul(s_p)
    return a_hat, xh_p[:N]


# speedup vs baseline: 8.6558x; 1.5038x over previous
import functools

import jax
import jax.numpy as jnp
from jax import lax
from jax.experimental import pallas as pl
from jax.experimental.pallas import tpu as pltpu
from jax.experimental.pallas import tpu_sc as plsc

N = 10000
D = 128
TRASH = 240
NP = N + TRASH
NTILES = 32
RPT = NP // 16
CHUNK = 128

@functools.cache
def _mesh():
    return plsc.VectorSubcoreMesh(core_axis_name="c", subcore_axis_name="s",
                                  num_cores=2, num_subcores=16)


def _agg_body(with_deg, streams, per_tile, *refs):
    if with_deg:
        (in_h, src_h, dst_h, z2_h, z1_h, one_h,
         p0_h, p1_h, d0_h, d1_h,
         acc, dacc, sidx, didx, rows0, rows1, ones_v, sem0, sem1) = refs
    else:
        (in_h, src_h, dst_h, z2_h,
         p0_h, p1_h,
         acc, sidx, didx, rows0, rows1, sem0, sem1) = refs
    c = lax.axis_index("c")
    s = lax.axis_index("s")
    rbase = s * RPT
    pltpu.sync_copy(z2_h, acc.at[pl.ds(rbase, RPT)])
    if with_deg:
        pltpu.sync_copy(z1_h, dacc.at[pl.ds(rbase, RPT)])
        pltpu.sync_copy(one_h, ones_v)

    tid = c * 16 + s
    plsc.subcore_barrier()

    bufs = (rows0, rows1)
    sems = (sem0, sem1)
    half = streams // 2

    def start_gather(k, b):
        pltpu.async_copy(in_h.at[sidx.at[k]], bufs[b], sems[b])

    def finish(k, b):
        pltpu.make_async_copy(in_h.at[sidx.at[k]], bufs[b], sems[b]).wait()
        pltpu.sync_copy(bufs[b], acc.at[didx.at[k]], add=True)
        if with_deg:
            pltpu.sync_copy(ones_v, dacc.at[didx.at[k]], add=True)

    for h in range(2):
        irow = tid * streams + h * half
        pltpu.sync_copy(src_h.at[pl.ds(irow, half)], sidx)
        pltpu.sync_copy(dst_h.at[pl.ds(irow, half)], didx)
        start_gather(0, 0)

        @pl.loop(0, half // 2)
        def _(g):
            k0 = 2 * g
            start_gather(k0 + 1, 1)
            finish(k0, 0)

            @pl.when(g < half // 2 - 1)
            def _():
                start_gather(k0 + 2, 0)

            finish(k0 + 1, 1)

    plsc.subcore_barrier()

    @pl.when(c == 0)
    def _():
        pltpu.sync_copy(acc.at[pl.ds(rbase, RPT)], p0_h.at[pl.ds(rbase, RPT)])
        if with_deg:
            pltpu.sync_copy(dacc.at[pl.ds(rbase, RPT)], d0_h.at[pl.ds(rbase, RPT)])

    @pl.when(c == 1)
    def _():
        pltpu.sync_copy(acc.at[pl.ds(rbase, RPT)], p1_h.at[pl.ds(rbase, RPT)])
        if with_deg:
            pltpu.sync_copy(dacc.at[pl.ds(rbase, RPT)], d1_h.at[pl.ds(rbase, RPT)])


def _make_agg(with_deg, streams, per_tile):
    f32 = jnp.float32
    out_type = [jax.ShapeDtypeStruct((NP, D), f32),
                jax.ShapeDtypeStruct((NP, D), f32)]
    scratch = [pltpu.VMEM_SHARED((NP, D), f32)]
    if with_deg:
        out_type += [jax.ShapeDtypeStruct((NP,), f32),
                     jax.ShapeDtypeStruct((NP,), f32)]
        scratch += [pltpu.VMEM_SHARED((NP,), f32)]
    scratch += [pltpu.VMEM((streams // 2, CHUNK), jnp.int32),
                pltpu.VMEM((streams // 2, CHUNK), jnp.int32),
                pltpu.VMEM((CHUNK, D), f32),
                pltpu.VMEM((CHUNK, D), f32)]
    if with_deg:
        scratch += [pltpu.VMEM((CHUNK,), f32)]
    scratch += [pltpu.SemaphoreType.DMA, pltpu.SemaphoreType.DMA]
    return pl.kernel(
        functools.partial(_agg_body, with_deg, streams, per_tile),
        out_type=out_type,
        mesh=_mesh(),
        scratch_types=scratch,
    )


def _combine_body(p0, p1, d0, d1, out):
    p = p0[...] + p1[...]
    d = d0[...] + d1[...]
    inv = 1.0 / jnp.maximum(d, 1.0)
    p3 = p.reshape(8, 128, D)
    out[...] = jnp.maximum(p3 * inv[:, :, None], 0.0).reshape(1024, D)


def _combine(P0, P1, d0r, d1r):
    return pl.pallas_call(
        _combine_body,
        grid=(NP // 1024,),
        in_specs=[
            pl.BlockSpec((1024, D), lambda i: (i, 0)),
            pl.BlockSpec((1024, D), lambda i: (i, 0)),
            pl.BlockSpec((8, 128), lambda i: (i, 0)),
            pl.BlockSpec((8, 128), lambda i: (i, 0)),
        ],
        out_specs=pl.BlockSpec((1024, D), lambda i: (i, 0)),
        out_shape=jax.ShapeDtypeStruct((NP, D), jnp.float32),
    )(P0, P1, d0r, d1r)


def _mm_body(a, b, out):
    out[...] = lax.dot_general(a[...], b[...], (((1,), (1,)), ((), ())),
                               preferred_element_type=jnp.float32)


def _matmul(s_p):
    bm, bn = 512, 512
    return pl.pallas_call(
        _mm_body,
        grid=(NP // bm, NP // bn),
        in_specs=[
            pl.BlockSpec((bm, D), lambda i, j: (i, 0)),
            pl.BlockSpec((bn, D), lambda i, j: (j, 0)),
        ],
        out_specs=pl.BlockSpec((bm, bn), lambda i, j: (i, j)),
        out_shape=jax.ShapeDtypeStruct((N, N), jnp.float32),
    )(s_p, s_p)


def kernel(x, edge_index):
    f32 = jnp.float32
    src = edge_index[0].astype(jnp.int32)
    dst = edge_index[1].astype(jnp.int32)
    e = src.shape[0]
    per = NTILES * CHUNK * 4
    ep = ((e + per - 1) // per) * per
    padn = ep - e
    pad_ids = jnp.arange(padn, dtype=jnp.int32)
    srcp = jnp.concatenate([src, (pad_ids * 37) % N]).reshape(ep // CHUNK, CHUNK)
    dstp = jnp.concatenate([dst, N + pad_ids % TRASH]).reshape(ep // CHUNK, CHUNK)
    per_tile = ep // NTILES
    streams = per_tile // CHUNK

    z2 = jnp.zeros((RPT, D), f32)
    z1 = jnp.zeros((RPT,), f32)
    one = jnp.ones((CHUNK,), f32)

    agg1 = _make_agg(True, streams, per_tile)
    agg = _make_agg(False, streams, per_tile)

    P0, P1, D0, D1 = agg1(x, srcp, dstp, z2, z1, one)
    d0r = D0.reshape(NP // 128, 128)
    d1r = D1.reshape(NP // 128, 128)
    h1 = _combine(P0, P1, d0r, d1r)
    Q0, Q1 = agg(h1, srcp, dstp, z2)
    h2 = _combine(Q0, Q1, d0r, d1r)
    R0, R1 = agg(h2, srcp, dstp, z2)
    s_p = _combine(R0, R1, d0r, d1r)
    T0, T1 = agg(s_p, srcp, dstp, z2)
    xh_p = _combine(T0, T1, d0r, d1r)
    a_hat = _matmul(s_p)
    return a_hat, xh_p[:N]
